# initial kernel scaffold (unmeasured)
import jax
import jax.numpy as jnp
from jax import lax
from jax.experimental import pallas as pl
from jax.experimental.pallas import tpu as pltpu

N_DEV = 8


def kernel(x, w_mat):
    m_per, k = x.shape
    _, n_per = w_mat.shape

    def body(x_ref, w_ref, out_ref, comm_ref, send_sems, recv_sems):
        my = lax.axis_index("i")
        left = lax.rem(my + N_DEV - 1, N_DEV)
        right = lax.rem(my + 1, N_DEV)

        barrier_sem = pltpu.get_barrier_semaphore()
        for nbr in (left, right):
            pl.semaphore_signal(
                barrier_sem, inc=1,
                device_id=(nbr,), device_id_type=pl.DeviceIdType.MESH,
            )
        pl.semaphore_wait(barrier_sem, 2)

        comm_ref[0, :, :] = x_ref[:, :]
        acc = jnp.dot(x_ref[:, :], w_ref[:, :], preferred_element_type=jnp.float32)
        out_ref[pl.ds(my * m_per, m_per), :] = jnp.maximum(acc, 0.0)

        for h in range(N_DEV - 1):
            send_slot = h % 2
            recv_slot = (h + 1) % 2
            rdma = pltpu.make_async_remote_copy(
                src_ref=comm_ref.at[send_slot],
                dst_ref=comm_ref.at[recv_slot],
                send_sem=send_sems.at[send_slot],
                recv_sem=recv_sems.at[recv_slot],
                device_id=(right,),
                device_id_type=pl.DeviceIdType.MESH,
            )
            rdma.start()
            rdma.wait()

            origin = lax.rem(my - h - 1 + N_DEV, N_DEV)
            acc = jnp.dot(
                comm_ref[recv_slot, :, :], w_ref[:, :],
                preferred_element_type=jnp.float32,
            )
            out_ref[pl.ds(origin * m_per, m_per), :] = jnp.maximum(acc, 0.0)

    return pl.pallas_call(
        body,
        out_shape=jax.ShapeDtypeStruct((N_DEV * m_per, n_per), jnp.float32),
        in_specs=[
            pl.BlockSpec(memory_space=pltpu.VMEM),
            pl.BlockSpec(memory_space=pltpu.VMEM),
        ],
        out_specs=pl.BlockSpec(memory_space=pltpu.VMEM),
        scratch_shapes=[
            pltpu.VMEM((2, m_per, k), x.dtype),
            pltpu.SemaphoreType.DMA((2,)),
            pltpu.SemaphoreType.DMA((2,)),
        ],
        compiler_params=pltpu.CompilerParams(collective_id=0),
    )(x, w_mat)


# baseline (device time: 408790 ns/iter reference)
import jax
import jax.numpy as jnp
from jax import lax
from jax.experimental import pallas as pl
from jax.experimental.pallas import tpu as pltpu

N_DEV = 8


def kernel(x, w_mat):
    x = x.astype(jnp.bfloat16)
    w_mat = w_mat.astype(jnp.bfloat16)
    m_per, k = x.shape
    _, n_per = w_mat.shape

    def body(x_ref, w_ref, out_ref, comm_ref, send_sems, recv_sems):
        my = lax.axis_index("i")
        left = lax.rem(my + N_DEV - 1, N_DEV)
        right = lax.rem(my + 1, N_DEV)

        barrier_sem = pltpu.get_barrier_semaphore()
        for nbr in (left, right):
            pl.semaphore_signal(
                barrier_sem, inc=1,
                device_id=(nbr,), device_id_type=pl.DeviceIdType.MESH,
            )
        pl.semaphore_wait(barrier_sem, 2)

        comm_ref[0, :, :] = x_ref[:, :]
        acc = jnp.dot(x_ref[:, :], w_ref[:, :], preferred_element_type=jnp.float32)
        out_ref[pl.ds(my * m_per, m_per), :] = jnp.maximum(acc, 0.0)

        for h in range(N_DEV - 1):
            send_slot = h % 2
            recv_slot = (h + 1) % 2
            rdma = pltpu.make_async_remote_copy(
                src_ref=comm_ref.at[send_slot],
                dst_ref=comm_ref.at[recv_slot],
                send_sem=send_sems.at[send_slot],
                recv_sem=recv_sems.at[recv_slot],
                device_id=(right,),
                device_id_type=pl.DeviceIdType.MESH,
            )
            rdma.start()
            rdma.wait()

            origin = lax.rem(my - h - 1 + N_DEV, N_DEV)
            acc = jnp.dot(
                comm_ref[recv_slot, :, :], w_ref[:, :],
                preferred_element_type=jnp.float32,
            )
            out_ref[pl.ds(origin * m_per, m_per), :] = jnp.maximum(acc, 0.0)

    return pl.pallas_call(
        body,
        out_shape=jax.ShapeDtypeStruct((N_DEV * m_per, n_per), jnp.float32),
        in_specs=[
            pl.BlockSpec(memory_space=pltpu.VMEM),
            pl.BlockSpec(memory_space=pltpu.VMEM),
        ],
        out_specs=pl.BlockSpec(memory_space=pltpu.VMEM),
        scratch_shapes=[
            pltpu.VMEM((2, m_per, k), x.dtype),
            pltpu.SemaphoreType.DMA((2,)),
            pltpu.SemaphoreType.DMA((2,)),
        ],
        compiler_params=pltpu.CompilerParams(
            collective_id=0, vmem_limit_bytes=60 * 1024 * 1024,
        ),
    )(x, w_mat)


# device time: 219688 ns/iter; 1.8608x vs baseline; 1.8608x over previous
import jax
import jax.numpy as jnp
from jax import lax
from jax.experimental import pallas as pl
from jax.experimental.pallas import tpu as pltpu

N_DEV = 8


def kernel(x, w_mat):
    x = x.astype(jnp.bfloat16)
    w_mat = w_mat.astype(jnp.bfloat16)
    m_per, k = x.shape
    _, n_per = w_mat.shape
    m_half = m_per // 2

    def body(x_ref, w_ref, out_ref, cw_ref, ccw_ref,
             cw_send_sems, cw_recv_sems, ccw_send_sems, ccw_recv_sems):
        my = lax.axis_index("i")
        left = lax.rem(my + N_DEV - 1, N_DEV)
        right = lax.rem(my + 1, N_DEV)

        barrier_sem = pltpu.get_barrier_semaphore()
        for nbr in (left, right):
            pl.semaphore_signal(
                barrier_sem, inc=1,
                device_id=(nbr,), device_id_type=pl.DeviceIdType.MESH,
            )
        pl.semaphore_wait(barrier_sem, 2)

        cw_ref[0, :, :] = x_ref[:m_half, :]
        ccw_ref[0, :, :] = x_ref[m_half:, :]

        def hop_rdmas(h):
            send_slot = h % 2
            recv_slot = (h + 1) % 2
            cw = pltpu.make_async_remote_copy(
                src_ref=cw_ref.at[send_slot],
                dst_ref=cw_ref.at[recv_slot],
                send_sem=cw_send_sems.at[send_slot],
                recv_sem=cw_recv_sems.at[recv_slot],
                device_id=(right,),
                device_id_type=pl.DeviceIdType.MESH,
            )
            ccw = pltpu.make_async_remote_copy(
                src_ref=ccw_ref.at[send_slot],
                dst_ref=ccw_ref.at[recv_slot],
                send_sem=ccw_send_sems.at[send_slot],
                recv_sem=ccw_recv_sems.at[recv_slot],
                device_id=(left,),
                device_id_type=pl.DeviceIdType.MESH,
            )
            return cw, ccw

        def gemm_halves(h, slot):
            cw_origin = lax.rem(my - h - 1 + N_DEV, N_DEV)
            ccw_origin = lax.rem(my + h + 1, N_DEV)
            acc = jnp.dot(cw_ref[slot, :, :], w_ref[:, :],
                          preferred_element_type=jnp.float32)
            out_ref[pl.ds(cw_origin * m_per, m_half), :] = jnp.maximum(acc, 0.0)
            acc = jnp.dot(ccw_ref[slot, :, :], w_ref[:, :],
                          preferred_element_type=jnp.float32)
            out_ref[pl.ds(ccw_origin * m_per + m_half, m_half), :] = (
                jnp.maximum(acc, 0.0))

        cw, ccw = hop_rdmas(0)
        cw.start()
        ccw.start()
        acc = jnp.dot(x_ref[:, :], w_ref[:, :],
                      preferred_element_type=jnp.float32)
        out_ref[pl.ds(my * m_per, m_per), :] = jnp.maximum(acc, 0.0)
        cw.wait()
        ccw.wait()

        for h in range(1, N_DEV - 1):
            cw, ccw = hop_rdmas(h)
            cw.start()
            ccw.start()
            gemm_halves(h - 1, h % 2)
            cw.wait()
            ccw.wait()

        gemm_halves(N_DEV - 2, (N_DEV - 1) % 2)

    return pl.pallas_call(
        body,
        out_shape=jax.ShapeDtypeStruct((N_DEV * m_per, n_per), jnp.float32),
        in_specs=[
            pl.BlockSpec(memory_space=pltpu.VMEM),
            pl.BlockSpec(memory_space=pltpu.VMEM),
        ],
        out_specs=pl.BlockSpec(memory_space=pltpu.VMEM),
        scratch_shapes=[
            pltpu.VMEM((2, m_half, k), x.dtype),
            pltpu.VMEM((2, m_half, k), x.dtype),
            pltpu.SemaphoreType.DMA((2,)),
            pltpu.SemaphoreType.DMA((2,)),
            pltpu.SemaphoreType.DMA((2,)),
            pltpu.SemaphoreType.DMA((2,)),
        ],
        compiler_params=pltpu.CompilerParams(
            collective_id=0, vmem_limit_bytes=60 * 1024 * 1024,
        ),
    )(x, w_mat)


# device time: 208461 ns/iter; 1.9610x vs baseline; 1.0539x over previous
import jax
import jax.numpy as jnp
from jax import lax
from jax.experimental import pallas as pl
from jax.experimental.pallas import tpu as pltpu

N_DEV = 8


def kernel(x, w_mat):
    m_per, k = x.shape
    _, n_per = w_mat.shape
    m_half = m_per // 2

    def body(x_ref, w_ref, out_ref, cw_ref, ccw_ref,
             cw_send_sems, cw_recv_sems, ccw_send_sems, ccw_recv_sems):
        my = lax.axis_index("i")
        left = lax.rem(my + N_DEV - 1, N_DEV)
        right = lax.rem(my + 1, N_DEV)

        barrier_sem = pltpu.get_barrier_semaphore()
        for nbr in (left, right):
            pl.semaphore_signal(
                barrier_sem, inc=1,
                device_id=(nbr,), device_id_type=pl.DeviceIdType.MESH,
            )
        pl.semaphore_wait(barrier_sem, 2)

        cw_ref[0, :, :] = x_ref[:m_half, :].astype(jnp.bfloat16)
        ccw_ref[0, :, :] = x_ref[m_half:, :].astype(jnp.bfloat16)

        def hop_rdmas(h):
            send_slot = h % 2
            recv_slot = (h + 1) % 2
            cw = pltpu.make_async_remote_copy(
                src_ref=cw_ref.at[send_slot],
                dst_ref=cw_ref.at[recv_slot],
                send_sem=cw_send_sems.at[send_slot],
                recv_sem=cw_recv_sems.at[recv_slot],
                device_id=(right,),
                device_id_type=pl.DeviceIdType.MESH,
            )
            ccw = pltpu.make_async_remote_copy(
                src_ref=ccw_ref.at[send_slot],
                dst_ref=ccw_ref.at[recv_slot],
                send_sem=ccw_send_sems.at[send_slot],
                recv_sem=ccw_recv_sems.at[recv_slot],
                device_id=(left,),
                device_id_type=pl.DeviceIdType.MESH,
            )
            return cw, ccw

        def gemm_halves(h, slot):
            cw_origin = lax.rem(my - h - 1 + N_DEV, N_DEV)
            ccw_origin = lax.rem(my + h + 1, N_DEV)
            acc = jnp.dot(cw_ref[slot, :, :], w_ref[:, :],
                          preferred_element_type=jnp.float32)
            out_ref[pl.ds(cw_origin * m_per, m_half), :] = jnp.maximum(acc, 0.0)
            acc = jnp.dot(ccw_ref[slot, :, :], w_ref[:, :],
                          preferred_element_type=jnp.float32)
            out_ref[pl.ds(ccw_origin * m_per + m_half, m_half), :] = (
                jnp.maximum(acc, 0.0))

        cw, ccw = hop_rdmas(0)
        cw.start()
        ccw.start()
        acc = jnp.dot(x_ref[:, :], w_ref[:, :],
                      preferred_element_type=jnp.float32)
        out_ref[pl.ds(my * m_per, m_per), :] = jnp.maximum(acc, 0.0)
        cw.wait()
        ccw.wait()

        for h in range(1, N_DEV - 1):
            cw, ccw = hop_rdmas(h)
            cw.start()
            ccw.start()
            gemm_halves(h - 1, h % 2)
            cw.wait()
            ccw.wait()

        gemm_halves(N_DEV - 2, (N_DEV - 1) % 2)

    return pl.pallas_call(
        body,
        out_shape=jax.ShapeDtypeStruct((N_DEV * m_per, n_per), jnp.float32),
        in_specs=[
            pl.BlockSpec(memory_space=pltpu.VMEM),
            pl.BlockSpec(memory_space=pltpu.VMEM),
        ],
        out_specs=pl.BlockSpec(memory_space=pltpu.VMEM),
        scratch_shapes=[
            pltpu.VMEM((2, m_half, k), jnp.bfloat16),
            pltpu.VMEM((2, m_half, k), jnp.bfloat16),
            pltpu.SemaphoreType.DMA((2,)),
            pltpu.SemaphoreType.DMA((2,)),
            pltpu.SemaphoreType.DMA((2,)),
            pltpu.SemaphoreType.DMA((2,)),
        ],
        compiler_params=pltpu.CompilerParams(
            collective_id=0, vmem_limit_bytes=60 * 1024 * 1024,
        ),
    )(x, w_mat)


# device time: 164157 ns/iter; 2.4902x vs baseline; 1.2699x over previous
import numpy as np

import jax
import jax.numpy as jnp
from jax import lax
from jax.experimental import pallas as pl
from jax.experimental.pallas import tpu as pltpu

N_DEV = 8
N_DIM = 3

_POS2C = [(0, 0, 0), (1, 0, 0), (1, 1, 0), (0, 1, 0),
          (0, 0, 1), (1, 0, 1), (1, 1, 1), (0, 1, 1)]
_C2POS = {c: i for i, c in enumerate(_POS2C)}


def _flip(pos, d):
    c = list(_POS2C[pos])
    c[d] ^= 1
    return _C2POS[tuple(c)]


_NBR = np.array([[_flip(l, d) for d in range(N_DIM)] for l in range(N_DEV)],
                dtype=np.int32)

_SEND = np.zeros((N_DEV, N_DIM, N_DIM, 4), dtype=np.int32)
_ARRV = np.zeros((N_DEV, N_DIM, N_DIM, 4), dtype=np.int32)
for l in range(N_DEV):
    for t in range(N_DIM):
        held = [l]
        for p in range(N_DIM):
            d = (t + p) % N_DIM
            held = sorted(held)
            arrv = sorted(_flip(o, d) for o in held)
            _SEND[l, p, t, :len(held)] = held
            _ARRV[l, p, t, :len(arrv)] = arrv
            held = held + arrv


def kernel(x, w_mat):
    x = x.astype(jnp.bfloat16)
    w_mat = w_mat.astype(jnp.bfloat16)
    m_per, k = x.shape
    _, n_per = w_mat.shape

    t_rows = (176, 176, 160)
    t_off = (0, 176, 352)

    me = lax.axis_index("i")
    nbrs = jnp.asarray(_NBR)[me]
    send_tbl = jnp.asarray(_SEND)[me]
    arrv_tbl = jnp.asarray(_ARRV)[me]

    def body(x_ref, w_ref, nbr_ref, send_ref, arrv_ref, out_ref,
             g_ref, send_sems, recv_sems):
        my = lax.axis_index("i")

        barrier_sem = pltpu.get_barrier_semaphore()
        for d in range(N_DIM):
            pl.semaphore_signal(
                barrier_sem, inc=1,
                device_id=(nbr_ref[d],), device_id_type=pl.DeviceIdType.MESH,
            )
        pl.semaphore_wait(barrier_sem, N_DIM)

        g_ref[my, :, :] = x_ref[:, :]

        def make_desc(p, t, j, o):
            d = (t + p) % N_DIM
            return pltpu.make_async_remote_copy(
                src_ref=g_ref.at[o, pl.ds(t_off[t], t_rows[t]), :],
                dst_ref=g_ref.at[o, pl.ds(t_off[t], t_rows[t]), :],
                send_sem=send_sems.at[p, t, j],
                recv_sem=recv_sems.at[p, t, j],
                device_id=(nbr_ref[d],),
                device_id_type=pl.DeviceIdType.MESH,
            )

        def start_sends(p):
            for t in range(N_DIM):
                for j in range(1 << p):
                    make_desc(p, t, j, send_ref[p, t, j]).start()

        def wait_recv(p, t, j):
            make_desc(p, t, j, arrv_ref[p, t, j]).wait_recv()

        def gemm_piece(p, t, j):
            o = arrv_ref[p, t, j]
            xs = g_ref[o, pl.ds(t_off[t], t_rows[t]), :]
            acc = jnp.dot(xs, w_ref[:, :], preferred_element_type=jnp.float32)
            out_ref[pl.ds(o * m_per + t_off[t], t_rows[t]), :] = (
                jnp.maximum(acc, 0.0))

        start_sends(0)
        acc = jnp.dot(x_ref[:, :], w_ref[:, :],
                      preferred_element_type=jnp.float32)
        out_ref[pl.ds(my * m_per, m_per), :] = jnp.maximum(acc, 0.0)
        for t in range(N_DIM):
            wait_recv(0, t, 0)

        start_sends(1)
        for t in range(N_DIM):
            gemm_piece(0, t, 0)
        for t in range(N_DIM):
            for j in range(2):
                wait_recv(1, t, j)

        start_sends(2)
        for t in range(N_DIM):
            for j in range(2):
                gemm_piece(1, t, j)
        for j in range(4):
            for t in range(N_DIM):
                wait_recv(2, t, j)
                gemm_piece(2, t, j)

        for p in range(N_DIM):
            for t in range(N_DIM):
                for j in range(1 << p):
                    make_desc(p, t, j, send_ref[p, t, j]).wait_send()

    return pl.pallas_call(
        body,
        out_shape=jax.ShapeDtypeStruct((N_DEV * m_per, n_per), jnp.float32),
        in_specs=[
            pl.BlockSpec(memory_space=pltpu.VMEM),
            pl.BlockSpec(memory_space=pltpu.VMEM),
            pl.BlockSpec(memory_space=pltpu.SMEM),
            pl.BlockSpec(memory_space=pltpu.SMEM),
            pl.BlockSpec(memory_space=pltpu.SMEM),
        ],
        out_specs=pl.BlockSpec(memory_space=pltpu.VMEM),
        scratch_shapes=[
            pltpu.VMEM((N_DEV, m_per, k), jnp.bfloat16),
            pltpu.SemaphoreType.DMA((N_DIM, N_DIM, 4)),
            pltpu.SemaphoreType.DMA((N_DIM, N_DIM, 4)),
        ],
        compiler_params=pltpu.CompilerParams(
            collective_id=0, vmem_limit_bytes=64 * 1024 * 1024,
        ),
    )(x, w_mat, nbrs, send_tbl, arrv_tbl)


# device time: 148787 ns/iter; 2.7475x vs baseline; 1.1033x over previous
import numpy as np

import jax
import jax.numpy as jnp
from jax import lax
from jax.experimental import pallas as pl
from jax.experimental.pallas import tpu as pltpu

N_DEV = 8
N_DIM = 3

_POS2C = [(0, 0, 0), (1, 0, 0), (1, 1, 0), (0, 1, 0),
          (0, 0, 1), (1, 0, 1), (1, 1, 1), (0, 1, 1)]
_C2POS = {c: i for i, c in enumerate(_POS2C)}


def _flip(pos, d):
    c = list(_POS2C[pos])
    c[d] ^= 1
    return _C2POS[tuple(c)]


_NBR = np.array([[_flip(l, d) for d in range(N_DIM)] for l in range(N_DEV)],
                dtype=np.int32)

_SEND = np.zeros((N_DEV, N_DIM, N_DIM, 4), dtype=np.int32)
_ARRV = np.zeros((N_DEV, N_DIM, N_DIM, 4), dtype=np.int32)
for l in range(N_DEV):
    for t in range(N_DIM):
        held = [l]
        for p in range(N_DIM):
            d = (t + p) % N_DIM
            held = sorted(held)
            arrv = sorted(_flip(o, d) for o in held)
            _SEND[l, p, t, :len(held)] = held
            _ARRV[l, p, t, :len(arrv)] = arrv
            held = held + arrv


def kernel(x, w_mat):
    m_per, k = x.shape
    _, n_per = w_mat.shape

    t_rows = (176, 176, 160)
    t_off = (0, 176, 352)
    stage_rows = max(t_rows)

    me = lax.axis_index("i")
    nbrs = jnp.asarray(_NBR)[me]
    send_tbl = jnp.asarray(_SEND)[me]
    arrv_tbl = jnp.asarray(_ARRV)[me]

    def body(x_ref, w_ref, nbr_ref, send_ref, arrv_ref, out_ref,
             g_ref, stage_ref, send_sems, recv_sems, copy_sems):
        my = lax.axis_index("i")

        barrier_sem = pltpu.get_barrier_semaphore()
        for d in range(N_DIM):
            pl.semaphore_signal(
                barrier_sem, inc=1,
                device_id=(nbr_ref[d],), device_id_type=pl.DeviceIdType.MESH,
            )
        pl.semaphore_wait(barrier_sem, N_DIM)

        g_ref[my, :, :] = x_ref[:, :].astype(jnp.bfloat16)

        def make_desc(p, t, j, o):
            d = (t + p) % N_DIM
            return pltpu.make_async_remote_copy(
                src_ref=g_ref.at[o, pl.ds(t_off[t], t_rows[t]), :],
                dst_ref=g_ref.at[o, pl.ds(t_off[t], t_rows[t]), :],
                send_sem=send_sems.at[p, t, j],
                recv_sem=recv_sems.at[p, t, j],
                device_id=(nbr_ref[d],),
                device_id_type=pl.DeviceIdType.MESH,
            )

        def start_sends(p):
            for t in range(N_DIM):
                for j in range(1 << p):
                    make_desc(p, t, j, send_ref[p, t, j]).start()

        def wait_recv(p, t, j):
            make_desc(p, t, j, arrv_ref[p, t, j]).wait_recv()

        pending = [None, None]
        counter = [0]

        def emit_piece(xs, rows, out_start):
            s = counter[0] % 2
            counter[0] += 1
            if pending[s] is not None:
                pending[s].wait()
            acc = jnp.dot(xs, w_ref[:, :], preferred_element_type=jnp.float32)
            stage_ref[s, pl.ds(0, rows), :] = jnp.maximum(acc, 0.0)
            cp = pltpu.make_async_copy(
                stage_ref.at[s, pl.ds(0, rows), :],
                out_ref.at[pl.ds(out_start, rows), :],
                copy_sems.at[s],
            )
            cp.start()
            pending[s] = cp

        def gemm_piece(p, t, j):
            o = arrv_ref[p, t, j]
            xs = g_ref[o, pl.ds(t_off[t], t_rows[t]), :]
            emit_piece(xs, t_rows[t], o * m_per + t_off[t])

        start_sends(0)
        for t in range(N_DIM):
            emit_piece(x_ref[pl.ds(t_off[t], t_rows[t]), :], t_rows[t],
                       my * m_per + t_off[t])
        for t in range(N_DIM):
            wait_recv(0, t, 0)

        start_sends(1)
        for t in range(N_DIM):
            gemm_piece(0, t, 0)
        for t in range(N_DIM):
            for j in range(2):
                wait_recv(1, t, j)

        start_sends(2)
        for t in range(N_DIM):
            for j in range(2):
                gemm_piece(1, t, j)
        for j in range(4):
            for t in range(N_DIM):
                wait_recv(2, t, j)
                gemm_piece(2, t, j)

        for s in (0, 1):
            if pending[s] is not None:
                pending[s].wait()
        for p in range(N_DIM):
            for t in range(N_DIM):
                for j in range(1 << p):
                    make_desc(p, t, j, send_ref[p, t, j]).wait_send()

    return pl.pallas_call(
        body,
        out_shape=jax.ShapeDtypeStruct((N_DEV * m_per, n_per), jnp.float32),
        in_specs=[
            pl.BlockSpec(memory_space=pltpu.VMEM),
            pl.BlockSpec(memory_space=pltpu.VMEM),
            pl.BlockSpec(memory_space=pltpu.SMEM),
            pl.BlockSpec(memory_space=pltpu.SMEM),
            pl.BlockSpec(memory_space=pltpu.SMEM),
        ],
        out_specs=pl.BlockSpec(memory_space=pltpu.MemorySpace.HBM),
        scratch_shapes=[
            pltpu.VMEM((N_DEV, m_per, k), jnp.bfloat16),
            pltpu.VMEM((2, stage_rows, n_per), jnp.float32),
            pltpu.SemaphoreType.DMA((N_DIM, N_DIM, 4)),
            pltpu.SemaphoreType.DMA((N_DIM, N_DIM, 4)),
            pltpu.SemaphoreType.DMA((2,)),
        ],
        compiler_params=pltpu.CompilerParams(
            collective_id=0, vmem_limit_bytes=64 * 1024 * 1024,
        ),
    )(x, w_mat, nbrs, send_tbl, arrv_tbl)


# device time: 147270 ns/iter; 2.7758x vs baseline; 1.0103x over previous
import numpy as np

import jax
import jax.numpy as jnp
from jax import lax
from jax.experimental import pallas as pl
from jax.experimental.pallas import tpu as pltpu

N_DEV = 8
N_DIM = 3

_POS2C = [(0, 0, 0), (1, 0, 0), (1, 1, 0), (0, 1, 0),
          (0, 0, 1), (1, 0, 1), (1, 1, 1), (0, 1, 1)]
_C2POS = {c: i for i, c in enumerate(_POS2C)}


def _flip(pos, d):
    c = list(_POS2C[pos])
    c[d] ^= 1
    return _C2POS[tuple(c)]


_NBR = np.array([[_flip(l, d) for d in range(N_DIM)] for l in range(N_DEV)],
                dtype=np.int32)

_SEND = np.zeros((N_DEV, N_DIM, N_DIM, 4), dtype=np.int32)
_ARRV = np.zeros((N_DEV, N_DIM, N_DIM, 4), dtype=np.int32)
for l in range(N_DEV):
    for t in range(N_DIM):
        held = [l]
        for p in range(N_DIM):
            d = (t + p) % N_DIM
            arrv = [_flip(o, d) for o in held]
            _SEND[l, p, t, :len(held)] = held
            _ARRV[l, p, t, :len(arrv)] = arrv
            held = held + arrv


def kernel(x, w_mat):
    m_per, k = x.shape
    _, n_per = w_mat.shape

    t_rows = (176, 176, 160)
    t_off = (0, 176, 352)
    stage_rows = max(t_rows)

    me = lax.axis_index("i")
    nbrs = jnp.asarray(_NBR)[me]
    send_tbl = jnp.asarray(_SEND)[me]
    arrv_tbl = jnp.asarray(_ARRV)[me]

    def body(x_ref, w_ref, nbr_ref, send_ref, arrv_ref, out_ref,
             g_ref, stage_ref, send_sems, recv_sems, copy_sems):
        my = lax.axis_index("i")

        barrier_sem = pltpu.get_barrier_semaphore()
        for d in range(N_DIM):
            pl.semaphore_signal(
                barrier_sem, inc=1,
                device_id=(nbr_ref[d],), device_id_type=pl.DeviceIdType.MESH,
            )
        pl.semaphore_wait(barrier_sem, N_DIM)

        g_ref[my, :, :] = x_ref[:, :].astype(jnp.bfloat16)

        def make_desc(p, t, j, o):
            d = (t + p) % N_DIM
            return pltpu.make_async_remote_copy(
                src_ref=g_ref.at[o, pl.ds(t_off[t], t_rows[t]), :],
                dst_ref=g_ref.at[o, pl.ds(t_off[t], t_rows[t]), :],
                send_sem=send_sems.at[p, t, j],
                recv_sem=recv_sems.at[p, t, j],
                device_id=(nbr_ref[d],),
                device_id_type=pl.DeviceIdType.MESH,
            )

        def start_send(p, t, j):
            make_desc(p, t, j, send_ref[p, t, j]).start()

        def wait_recv(p, t, j):
            make_desc(p, t, j, arrv_ref[p, t, j]).wait_recv()

        pending = [None, None]
        counter = [0]

        def emit_piece(xs, rows, out_start):
            s = counter[0] % 2
            counter[0] += 1
            if pending[s] is not None:
                pending[s].wait()
            acc = jnp.dot(xs, w_ref[:, :], preferred_element_type=jnp.float32)
            stage_ref[s, pl.ds(0, rows), :] = jnp.maximum(acc, 0.0)
            cp = pltpu.make_async_copy(
                stage_ref.at[s, pl.ds(0, rows), :],
                out_ref.at[pl.ds(out_start, rows), :],
                copy_sems.at[s],
            )
            cp.start()
            pending[s] = cp

        def gemm_piece(p, t, j):
            o = arrv_ref[p, t, j]
            xs = g_ref[o, pl.ds(t_off[t], t_rows[t]), :]
            emit_piece(xs, t_rows[t], o * m_per + t_off[t])

        for t in range(N_DIM):
            start_send(0, t, 0)
            start_send(1, t, 0)
            start_send(2, t, 0)
        for t in range(N_DIM):
            emit_piece(x_ref[pl.ds(t_off[t], t_rows[t]), :], t_rows[t],
                       my * m_per + t_off[t])

        for t in range(N_DIM):
            wait_recv(0, t, 0)
            start_send(1, t, 1)
            start_send(2, t, 1)
        for t in range(N_DIM):
            gemm_piece(0, t, 0)

        for t in range(N_DIM):
            wait_recv(1, t, 0)
            start_send(2, t, 2)
        for t in range(N_DIM):
            wait_recv(1, t, 1)
            start_send(2, t, 3)
        for t in range(N_DIM):
            for j in range(2):
                gemm_piece(1, t, j)

        for j in range(4):
            for t in range(N_DIM):
                wait_recv(2, t, j)
                gemm_piece(2, t, j)

        for s in (0, 1):
            if pending[s] is not None:
                pending[s].wait()
        for p in range(N_DIM):
            for t in range(N_DIM):
                for j in range(1 << p):
                    make_desc(p, t, j, send_ref[p, t, j]).wait_send()

    return pl.pallas_call(
        body,
        out_shape=jax.ShapeDtypeStruct((N_DEV * m_per, n_per), jnp.float32),
        in_specs=[
            pl.BlockSpec(memory_space=pltpu.VMEM),
            pl.BlockSpec(memory_space=pltpu.VMEM),
            pl.BlockSpec(memory_space=pltpu.SMEM),
            pl.BlockSpec(memory_space=pltpu.SMEM),
            pl.BlockSpec(memory_space=pltpu.SMEM),
        ],
        out_specs=pl.BlockSpec(memory_space=pltpu.MemorySpace.HBM),
        scratch_shapes=[
            pltpu.VMEM((N_DEV, m_per, k), jnp.bfloat16),
            pltpu.VMEM((2, stage_rows, n_per), jnp.float32),
            pltpu.SemaphoreType.DMA((N_DIM, N_DIM, 4)),
            pltpu.SemaphoreType.DMA((N_DIM, N_DIM, 4)),
            pltpu.SemaphoreType.DMA((2,)),
        ],
        compiler_params=pltpu.CompilerParams(
            collective_id=0, vmem_limit_bytes=64 * 1024 * 1024,
        ),
    )(x, w_mat, nbrs, send_tbl, arrv_tbl)


# device time: 138543 ns/iter; 2.9506x vs baseline; 1.0630x over previous
import numpy as np

import jax
import jax.numpy as jnp
from jax import lax
from jax.experimental import pallas as pl
from jax.experimental.pallas import tpu as pltpu

N_DEV = 8
N_DIM = 3

_POS2C = [(0, 0, 0), (1, 0, 0), (1, 1, 0), (0, 1, 0),
          (0, 0, 1), (1, 0, 1), (1, 1, 1), (0, 1, 1)]
_C2POS = {c: i for i, c in enumerate(_POS2C)}


def _flip(pos, d):
    c = list(_POS2C[pos])
    c[d] ^= 1
    return _C2POS[tuple(c)]


_NBR = np.array([[_flip(l, d) for d in range(N_DIM)] for l in range(N_DEV)],
                dtype=np.int32)

_SEND = np.zeros((N_DEV, N_DIM, N_DIM, 4), dtype=np.int32)
_ARRV = np.zeros((N_DEV, N_DIM, N_DIM, 4), dtype=np.int32)
for l in range(N_DEV):
    for t in range(N_DIM):
        held = [l]
        for p in range(N_DIM):
            d = (t + p) % N_DIM
            arrv = [_flip(o, d) for o in held]
            _SEND[l, p, t, :len(held)] = held
            _ARRV[l, p, t, :len(arrv)] = arrv
            held = held + arrv


def kernel(x, w_mat):
    m_per, k = x.shape
    _, n_per = w_mat.shape

    t_rows = (176, 176, 160)
    t_off = (0, 176, 352)
    stage_rows = max(t_rows)

    me = lax.axis_index("i")
    nbrs = jnp.asarray(_NBR)[me]
    send_tbl = jnp.asarray(_SEND)[me]
    arrv_tbl = jnp.asarray(_ARRV)[me]

    n_wchunk = 4
    wc_rows = k // n_wchunk

    def body(x_ref, w_hbm_ref, nbr_ref, send_ref, arrv_ref, out_ref,
             g_ref, w_ref, wtmp_ref, stage_ref,
             send_sems, recv_sems, copy_sems, wcopy_sems):
        my = lax.axis_index("i")

        barrier_sem = pltpu.get_barrier_semaphore()
        for d in range(N_DIM):
            pl.semaphore_signal(
                barrier_sem, inc=1,
                device_id=(nbr_ref[d],), device_id_type=pl.DeviceIdType.MESH,
            )
        pl.semaphore_wait(barrier_sem, N_DIM)

        g_ref[my, :, :] = x_ref[:, :].astype(jnp.bfloat16)

        def w_chunk_copy(c):
            return pltpu.make_async_copy(
                w_hbm_ref.at[pl.ds(c * wc_rows, wc_rows), :],
                wtmp_ref.at[c % 2],
                wcopy_sems.at[c % 2],
            )

        def convert_w():
            w_chunk_copy(0).start()
            for c in range(n_wchunk):
                if c + 1 < n_wchunk:
                    w_chunk_copy(c + 1).start()
                w_chunk_copy(c).wait()
                w_ref[pl.ds(c * wc_rows, wc_rows), :] = (
                    wtmp_ref[c % 2].astype(jnp.bfloat16))

        def make_desc(p, t, j, o):
            d = (t + p) % N_DIM
            return pltpu.make_async_remote_copy(
                src_ref=g_ref.at[o, pl.ds(t_off[t], t_rows[t]), :],
                dst_ref=g_ref.at[o, pl.ds(t_off[t], t_rows[t]), :],
                send_sem=send_sems.at[p, t, j],
                recv_sem=recv_sems.at[p, t, j],
                device_id=(nbr_ref[d],),
                device_id_type=pl.DeviceIdType.MESH,
            )

        def start_send(p, t, j):
            make_desc(p, t, j, send_ref[p, t, j]).start()

        def wait_recv(p, t, j):
            make_desc(p, t, j, arrv_ref[p, t, j]).wait_recv()

        pending = [None, None]
        counter = [0]

        def emit_piece(xs, rows, out_start):
            s = counter[0] % 2
            counter[0] += 1
            if pending[s] is not None:
                pending[s].wait()
            acc = jnp.dot(xs, w_ref[:, :], preferred_element_type=jnp.float32)
            stage_ref[s, pl.ds(0, rows), :] = jnp.maximum(acc, 0.0)
            cp = pltpu.make_async_copy(
                stage_ref.at[s, pl.ds(0, rows), :],
                out_ref.at[pl.ds(out_start, rows), :],
                copy_sems.at[s],
            )
            cp.start()
            pending[s] = cp

        def gemm_piece(p, t, j):
            o = arrv_ref[p, t, j]
            xs = g_ref[o, pl.ds(t_off[t], t_rows[t]), :]
            emit_piece(xs, t_rows[t], o * m_per + t_off[t])

        for t in range(N_DIM):
            start_send(0, t, 0)
            start_send(1, t, 0)
            start_send(2, t, 0)
        convert_w()
        for t in range(N_DIM):
            emit_piece(g_ref[my, pl.ds(t_off[t], t_rows[t]), :], t_rows[t],
                       my * m_per + t_off[t])

        for t in range(N_DIM):
            wait_recv(0, t, 0)
            start_send(1, t, 1)
            start_send(2, t, 1)
        for t in range(N_DIM):
            gemm_piece(0, t, 0)

        for t in range(N_DIM):
            wait_recv(1, t, 0)
            start_send(2, t, 2)
        for t in range(N_DIM):
            wait_recv(1, t, 1)
            start_send(2, t, 3)
        for t in range(N_DIM):
            for j in range(2):
                gemm_piece(1, t, j)

        for j in range(4):
            for t in range(N_DIM):
                wait_recv(2, t, j)
                gemm_piece(2, t, j)

        for s in (0, 1):
            if pending[s] is not None:
                pending[s].wait()
        for p in range(N_DIM):
            for t in range(N_DIM):
                for j in range(1 << p):
                    make_desc(p, t, j, send_ref[p, t, j]).wait_send()

    return pl.pallas_call(
        body,
        out_shape=jax.ShapeDtypeStruct((N_DEV * m_per, n_per), jnp.float32),
        in_specs=[
            pl.BlockSpec(memory_space=pltpu.VMEM),
            pl.BlockSpec(memory_space=pltpu.MemorySpace.HBM),
            pl.BlockSpec(memory_space=pltpu.SMEM),
            pl.BlockSpec(memory_space=pltpu.SMEM),
            pl.BlockSpec(memory_space=pltpu.SMEM),
        ],
        out_specs=pl.BlockSpec(memory_space=pltpu.MemorySpace.HBM),
        scratch_shapes=[
            pltpu.VMEM((N_DEV, m_per, k), jnp.bfloat16),
            pltpu.VMEM((k, n_per), jnp.bfloat16),
            pltpu.VMEM((2, k // n_wchunk, n_per), jnp.float32),
            pltpu.VMEM((2, stage_rows, n_per), jnp.float32),
            pltpu.SemaphoreType.DMA((N_DIM, N_DIM, 4)),
            pltpu.SemaphoreType.DMA((N_DIM, N_DIM, 4)),
            pltpu.SemaphoreType.DMA((2,)),
            pltpu.SemaphoreType.DMA((2,)),
        ],
        compiler_params=pltpu.CompilerParams(
            collective_id=0, vmem_limit_bytes=64 * 1024 * 1024,
        ),
    )(x, w_mat, nbrs, send_tbl, arrv_tbl)
